# Initial kernel scaffold; baseline (speedup 1.0000x reference)
import jax
import jax.numpy as jnp
from jax.experimental import pallas as pl


def kernel(x, prob, mag, temp):
    raise NotImplementedError("write your pallas kernel here")


# speedup vs baseline: 5.0272x; 5.0272x over previous
import jax
import jax.numpy as jnp
from jax import lax
from jax.experimental import pallas as pl
from jax.experimental.pallas import tpu as pltpu

_NB_TF = 4
_N_SEQ_TF = 2


def _body(s_ref, f_ref, x_ref, o_ref):
    i = pl.program_id(0)
    s0 = s_ref[0, i]
    s1 = s_ref[1, i]
    f = f_ref[0]
    x = x_ref[...]
    n = x.size
    inf = jnp.float32(jnp.inf)
    one = jnp.float32(1.0)
    zero = jnp.float32(0.0)

    m0 = jnp.sum(x) / n
    c0 = s0 >= 2
    a0 = jnp.where(c0, f, one)
    b0 = jnp.where(s0 == 3, m0 * (one - f), zero)
    lo0 = jnp.where(c0, zero, -inf)
    hi0 = jnp.where(c0, one, inf)
    y = jnp.minimum(jnp.maximum(x * a0 + b0, lo0), hi0)

    m1 = jnp.sum(y) / n
    c1 = s1 >= 2
    a1 = jnp.where(c1, f, one)
    b1 = jnp.where(s1 == 3, m1 * (one - f), zero)
    lo1 = jnp.where(c1, zero, -inf)
    hi1 = jnp.where(c1, one, inf)
    z = jnp.minimum(jnp.maximum(y * a1 + b1, lo1), hi1)

    flip = (s0 == 1) != (s1 == 1)
    w = x.shape[3]
    rows = lax.broadcasted_iota(jnp.int32, (w, w), 0)
    cols = lax.broadcasted_iota(jnp.int32, (w, w), 1)
    r = jnp.where(rows + cols == w - 1, one, zero)
    z2 = z.reshape(x.shape[1] * x.shape[2], w)
    zf = jnp.dot(z2, r, preferred_element_type=jnp.float32).reshape(x.shape)
    o_ref[...] = jnp.where(flip, zf, z)


def kernel(x, prob, mag, temp):
    batch = x.shape[0]
    temp_d = lax.stop_gradient(temp)
    mag_d = lax.stop_gradient(mag)
    distrib = jax.nn.softmax(prob * temp_d, axis=0)
    logits = jnp.log(distrib + 1e-12)
    skey = jax.random.key(42)
    samples = jax.random.categorical(
        skey, jnp.broadcast_to(logits, (batch, _NB_TF)), axis=-1,
        shape=(_N_SEQ_TF, batch)
    ).astype(jnp.int32)
    f = (jnp.float32(0.5) + mag_d / jnp.float32(1.0)).reshape((1,))

    out = pl.pallas_call(
        _body,
        grid=(batch,),
        in_specs=[
            pl.BlockSpec(memory_space=pltpu.SMEM),
            pl.BlockSpec(memory_space=pltpu.SMEM),
            pl.BlockSpec((1,) + x.shape[1:], lambda i: (i, 0, 0, 0)),
        ],
        out_specs=pl.BlockSpec((1,) + x.shape[1:], lambda i: (i, 0, 0, 0)),
        out_shape=jax.ShapeDtypeStruct(x.shape, x.dtype),
    )(samples, f, x)
    return out


# speedup vs baseline: 6.2338x; 1.2400x over previous
import jax
import jax.numpy as jnp
from jax import lax
from jax.experimental import pallas as pl
from jax.experimental.pallas import tpu as pltpu

_NB_TF = 4
_N_SEQ_TF = 2
_BLK = 4


def _body(c_ref, x_ref, o_ref):
    i = pl.program_id(0)
    ch, h, w = x_ref.shape[1], x_ref.shape[2], x_ref.shape[3]
    n = ch * h * w
    rows = lax.broadcasted_iota(jnp.int32, (w, w), 0)
    cols = lax.broadcasted_iota(jnp.int32, (w, w), 1)
    anti = rows + cols == w - 1
    diag = rows == cols
    for b in range(_BLK):
        col = i * _BLK + b
        a0 = c_ref[0, col]
        g0 = c_ref[1, col]
        lo0 = c_ref[2, col]
        hi0 = c_ref[3, col]
        a1 = c_ref[4, col]
        g1 = c_ref[5, col]
        lo1 = c_ref[6, col]
        hi1 = c_ref[7, col]
        fb = c_ref[8, col]
        xb = x_ref[b].reshape(ch * h, w)
        m0 = jnp.sum(xb) / n
        y = jnp.minimum(jnp.maximum(xb * a0 + g0 * m0, lo0), hi0)
        m1 = jnp.sum(y) / n
        z = jnp.minimum(jnp.maximum(y * a1 + g1 * m1, lo1), hi1)
        perm = jnp.where(anti, fb, 0.0) + jnp.where(diag, 1.0 - fb, 0.0)
        o_ref[b] = jnp.dot(z, perm, preferred_element_type=jnp.float32
                           ).reshape(ch, h, w)


def kernel(x, prob, mag, temp):
    batch = x.shape[0]
    temp_d = lax.stop_gradient(temp)
    mag_d = lax.stop_gradient(mag)
    distrib = jax.nn.softmax(prob * temp_d, axis=0)
    logits = jnp.log(distrib + 1e-12)
    skey = jax.random.key(42)
    samples = jax.random.categorical(
        skey, jnp.broadcast_to(logits, (batch, _NB_TF)), axis=-1,
        shape=(_N_SEQ_TF, batch)
    ).astype(jnp.int32)
    s0, s1 = samples[0], samples[1]

    f = jnp.float32(0.5) + mag_d / jnp.float32(1.0)
    one = jnp.float32(1.0)
    zero = jnp.float32(0.0)
    inf = jnp.float32(jnp.inf)

    def coeffs(s):
        c = s >= 2
        a = jnp.where(c, f, one)
        g = jnp.where(s == 3, one - f, zero)
        lo = jnp.where(c, zero, -inf)
        hi = jnp.where(c, one, inf)
        return a, g, lo, hi

    a0, g0, lo0, hi0 = coeffs(s0)
    a1, g1, lo1, hi1 = coeffs(s1)
    flip = ((s0 == 1) != (s1 == 1)).astype(jnp.float32)
    ctab = jnp.stack([a0, g0, lo0, hi0, a1, g1, lo1, hi1, flip], axis=0)

    out = pl.pallas_call(
        _body,
        grid=(batch // _BLK,),
        in_specs=[
            pl.BlockSpec(memory_space=pltpu.SMEM),
            pl.BlockSpec((_BLK,) + x.shape[1:], lambda i: (i, 0, 0, 0)),
        ],
        out_specs=pl.BlockSpec((_BLK,) + x.shape[1:], lambda i: (i, 0, 0, 0)),
        out_shape=jax.ShapeDtypeStruct(x.shape, x.dtype),
    )(ctab, x)
    return out


# speedup vs baseline: 6.2878x; 1.0087x over previous
import jax
import jax.numpy as jnp
from jax import lax
from jax.experimental import pallas as pl
from jax.experimental.pallas import tpu as pltpu

_NB_TF = 4
_N_SEQ_TF = 2
_BLK = 4


def _body(c_ref, x_ref, o_ref):
    i = pl.program_id(0)
    ch, h, w = x_ref.shape[1], x_ref.shape[2], x_ref.shape[3]
    n = ch * h * w
    rows = lax.broadcasted_iota(jnp.int32, (w, w), 0)
    cols = lax.broadcasted_iota(jnp.int32, (w, w), 1)
    anti = rows + cols == w - 1
    diag = rows == cols
    for b in range(_BLK):
        col = i * _BLK + b
        a0 = c_ref[0, col]
        g0 = c_ref[1, col]
        lo0 = c_ref[2, col]
        hi0 = c_ref[3, col]
        a1 = c_ref[4, col]
        g1 = c_ref[5, col]
        lo1 = c_ref[6, col]
        hi1 = c_ref[7, col]
        fb = c_ref[8, col]
        xb = x_ref[b].reshape(ch * h, w)
        perm = jnp.where(anti, fb, 0.0) + jnp.where(diag, 1.0 - fb, 0.0)
        xf = jnp.dot(xb, perm, preferred_element_type=jnp.float32)
        ones_row = jnp.full((8, ch * h), 1.0, dtype=jnp.float32)
        m0 = jnp.sum(jnp.dot(ones_row, xb,
                             preferred_element_type=jnp.float32)[0]) / n
        y = jnp.minimum(jnp.maximum(xf * a0 + g0 * m0, lo0), hi0)
        m1 = jnp.sum(jnp.dot(ones_row, y,
                             preferred_element_type=jnp.float32)[0]) / n
        z = jnp.minimum(jnp.maximum(y * a1 + g1 * m1, lo1), hi1)
        o_ref[b] = z.reshape(ch, h, w)


def kernel(x, prob, mag, temp):
    batch = x.shape[0]
    temp_d = lax.stop_gradient(temp)
    mag_d = lax.stop_gradient(mag)
    distrib = jax.nn.softmax(prob * temp_d, axis=0)
    logits = jnp.log(distrib + 1e-12)
    skey = jax.random.key(42)
    samples = jax.random.categorical(
        skey, jnp.broadcast_to(logits, (batch, _NB_TF)), axis=-1,
        shape=(_N_SEQ_TF, batch)
    ).astype(jnp.int32)
    s0, s1 = samples[0], samples[1]

    f = jnp.float32(0.5) + mag_d / jnp.float32(1.0)
    one = jnp.float32(1.0)
    zero = jnp.float32(0.0)
    inf = jnp.float32(jnp.inf)

    def coeffs(s):
        c = s >= 2
        a = jnp.where(c, f, one)
        g = jnp.where(s == 3, one - f, zero)
        lo = jnp.where(c, zero, -inf)
        hi = jnp.where(c, one, inf)
        return a, g, lo, hi

    a0, g0, lo0, hi0 = coeffs(s0)
    a1, g1, lo1, hi1 = coeffs(s1)
    flip = ((s0 == 1) != (s1 == 1)).astype(jnp.float32)
    ctab = jnp.stack([a0, g0, lo0, hi0, a1, g1, lo1, hi1, flip], axis=0)

    out = pl.pallas_call(
        _body,
        grid=(batch // _BLK,),
        in_specs=[
            pl.BlockSpec(memory_space=pltpu.SMEM),
            pl.BlockSpec((_BLK,) + x.shape[1:], lambda i: (i, 0, 0, 0)),
        ],
        out_specs=pl.BlockSpec((_BLK,) + x.shape[1:], lambda i: (i, 0, 0, 0)),
        out_shape=jax.ShapeDtypeStruct(x.shape, x.dtype),
    )(ctab, x)
    return out
